# baseline (speedup 1.0000x reference)
import functools

import jax
import jax.numpy as jnp
from jax import lax
from jax.experimental import pallas as pl
from jax.experimental.pallas import tpu as pltpu
from jax.experimental.pallas import tpu_sc as plsc


def _stage1_body(x_ref, w1t_ref, b1_ref, w2t_ref, b2_ref,
                 pmax_ref, amax_ref, col_ref):
    xb = x_ref[...]
    h = jnp.maximum(
        jnp.dot(xb, w1t_ref[...], preferred_element_type=jnp.float32)
        + b1_ref[...], 0.0)
    logits = (jnp.dot(h, w2t_ref[...], preferred_element_type=jnp.float32)
              + b2_ref[...])
    m = jnp.max(logits, axis=1, keepdims=True)
    e = jnp.exp(logits - m)
    s = jnp.sum(e, axis=1, keepdims=True)
    p = e / s
    br, ne = p.shape
    amax = jnp.argmax(p, axis=1).astype(jnp.int32)[:, None]
    onehot = jax.lax.broadcasted_iota(jnp.int32, (br, ne), 1) == amax
    masked = jnp.where(onehot, p, 0.0)
    pmax_ref[...] = jnp.max(p, axis=1, keepdims=True)
    amax_ref[...] = amax
    col_ref[...] = jnp.sum(masked, axis=0)[None, None, :]


def _sc_expand_body(pmax_hbm, amax_hbm, col_hbm, out_hbm,
                    colv, denom_v, pmax_v, amax_v, buf,
                    *, nc, nb, ne, rpw, capacity):
    wid = lax.axis_index("s") * nc + lax.axis_index("c")
    base = wid * rpw
    pltpu.sync_copy(col_hbm, colv)
    pltpu.sync_copy(pmax_hbm.at[pl.ds(base, rpw)], pmax_v)
    pltpu.sync_copy(amax_hbm.at[pl.ds(base, rpw)], amax_v)
    for q in range(ne // 16):
        acc = jnp.full((16,), 0.0001, dtype=jnp.float32)
        for j in range(nb):
            acc = acc + colv[pl.ds(j * ne + q * 16, 16)]
        denom_v[pl.ds(q * 16, 16)] = acc

    ch = buf.shape[0]

    def _zero_row(r, carry):
        for q in range(ne // 16):
            buf[r, pl.ds(q * 16, 16)] = jnp.zeros((16,), jnp.float32)
        return carry

    for c in range(rpw // ch):
        lax.fori_loop(0, ch, _zero_row, 0)
        for t in range(ch // 16):
            off = c * ch + t * 16
            idx16 = amax_v[pl.ds(off, 16)]
            p16 = pmax_v[pl.ds(off, 16)]
            d16 = plsc.load_gather(denom_v, [idx16])
            val16 = (p16 / d16) * capacity
            row16 = lax.iota(jnp.int32, 16) + t * 16
            plsc.store_scatter(buf, [row16, idx16], val16)
        pltpu.sync_copy(buf, out_hbm.at[pl.ds(base + c * ch, ch)])


def kernel(x, W1, b1, W2, b2):
    n, d = x.shape
    nh = W1.shape[0]
    ne = W2.shape[0]
    br = 4096
    nb = n // br
    capacity = float(n)

    w1t = W1.T
    w2t = W2.T
    b1r = b1.reshape(1, nh)
    b2r = b2.reshape(1, ne)

    pmax, amax, colpart = pl.pallas_call(
        _stage1_body,
        grid=(nb,),
        in_specs=[
            pl.BlockSpec((br, d), lambda i: (i, 0)),
            pl.BlockSpec((d, nh), lambda i: (0, 0)),
            pl.BlockSpec((1, nh), lambda i: (0, 0)),
            pl.BlockSpec((nh, ne), lambda i: (0, 0)),
            pl.BlockSpec((1, ne), lambda i: (0, 0)),
        ],
        out_specs=[
            pl.BlockSpec((br, 1), lambda i: (i, 0)),
            pl.BlockSpec((br, 1), lambda i: (i, 0)),
            pl.BlockSpec((1, 1, ne), lambda i: (i, 0, 0)),
        ],
        out_shape=[
            jax.ShapeDtypeStruct((n, 1), jnp.float32),
            jax.ShapeDtypeStruct((n, 1), jnp.int32),
            jax.ShapeDtypeStruct((nb, 1, ne), jnp.float32),
        ],
        compiler_params=pltpu.CompilerParams(
            dimension_semantics=("parallel",)),
    )(x, w1t, b1r, w2t, b2r)

    info = plsc.get_sparse_core_info()
    nc, ns = info.num_cores, info.num_subcores
    rpw = n // (nc * ns)
    mesh = plsc.VectorSubcoreMesh(core_axis_name="c", subcore_axis_name="s")
    out = pl.kernel(
        functools.partial(_sc_expand_body, nc=nc, nb=nb, ne=ne, rpw=rpw,
                          capacity=capacity),
        out_type=jax.ShapeDtypeStruct((n, ne), jnp.float32),
        mesh=mesh,
        compiler_params=pltpu.CompilerParams(needs_layout_passes=False),
        scratch_types=[
            pltpu.VMEM((nb * ne,), jnp.float32),
            pltpu.VMEM((128,), jnp.float32),
            pltpu.VMEM((rpw,), jnp.float32),
            pltpu.VMEM((rpw,), jnp.int32),
            pltpu.VMEM((256, ne), jnp.float32),
        ],
    )(pmax.reshape(n), amax.reshape(n), colpart.reshape(nb * ne))

    return out


# speedup vs baseline: 1.3649x; 1.3649x over previous
import functools

import jax
import jax.numpy as jnp
from jax.experimental import pallas as pl
from jax.experimental.pallas import tpu as pltpu


def _stage1_body(xa_ref, xb_ref, w1ta_ref, w1tb_ref, b1_ref, w2t_ref, b2_ref,
                 pmax_ref, amax_ref, col_ref):
    h = jnp.maximum(
        jnp.dot(xa_ref[...], w1ta_ref[...], preferred_element_type=jnp.float32)
        + jnp.dot(xb_ref[...], w1tb_ref[...], preferred_element_type=jnp.float32)
        + b1_ref[...], 0.0)
    logits = (jnp.dot(h, w2t_ref[...], preferred_element_type=jnp.float32)
              + b2_ref[...])
    m = jnp.max(logits, axis=1, keepdims=True)
    e = jnp.exp(logits - m)
    s = jnp.sum(e, axis=1, keepdims=True)
    p = e / s
    br, ne = p.shape
    amax = jnp.argmax(p, axis=1).astype(jnp.int32)[:, None]
    onehot = jax.lax.broadcasted_iota(jnp.int32, (br, ne), 1) == amax
    masked = jnp.where(onehot, p, 0.0)
    pmax_ref[...] = jnp.max(p, axis=1, keepdims=True)
    amax_ref[...] = amax
    col_ref[...] = jnp.sum(masked, axis=0)[None, None, :]


def _stage2_body(pmax_ref, amax_ref, col_ref, out_ref, *, capacity):
    cols = col_ref[...]
    denom = jnp.sum(cols, axis=(0, 1))[None, :] + 0.0001
    t = (pmax_ref[...] / denom) * capacity
    br, ne = t.shape
    onehot = (jax.lax.broadcasted_iota(jnp.int32, (br, ne), 1)
              == amax_ref[...])
    out_ref[...] = jnp.where(onehot, t, 0.0)


def kernel(x, W1, b1, W2, b2):
    n, d = x.shape
    nh = W1.shape[0]
    ne = W2.shape[0]
    br = 4096
    nb = n // br
    dh = d // 2
    capacity = float(n)

    w1t = W1.T
    w1ta = w1t[:dh]
    w1tb = w1t[dh:]
    w2t = W2.T
    b1r = b1.reshape(1, nh)
    b2r = b2.reshape(1, ne)

    pmax, amax, colpart = pl.pallas_call(
        _stage1_body,
        grid=(nb,),
        in_specs=[
            pl.BlockSpec((br, dh), lambda i: (i, 0)),
            pl.BlockSpec((br, dh), lambda i: (i, 1)),
            pl.BlockSpec((dh, nh), lambda i: (0, 0)),
            pl.BlockSpec((dh, nh), lambda i: (0, 0)),
            pl.BlockSpec((1, nh), lambda i: (0, 0)),
            pl.BlockSpec((nh, ne), lambda i: (0, 0)),
            pl.BlockSpec((1, ne), lambda i: (0, 0)),
        ],
        out_specs=[
            pl.BlockSpec((br, 1), lambda i: (i, 0)),
            pl.BlockSpec((br, 1), lambda i: (i, 0)),
            pl.BlockSpec((1, 1, ne), lambda i: (i, 0, 0)),
        ],
        out_shape=[
            jax.ShapeDtypeStruct((n, 1), jnp.float32),
            jax.ShapeDtypeStruct((n, 1), jnp.int32),
            jax.ShapeDtypeStruct((nb, 1, ne), jnp.float32),
        ],
        compiler_params=pltpu.CompilerParams(
            dimension_semantics=("parallel",)),
    )(x, x, w1ta, w1tb, b1r, w2t, b2r)

    br2 = 16384
    nb2 = n // br2
    out = pl.pallas_call(
        functools.partial(_stage2_body, capacity=capacity),
        grid=(nb2,),
        in_specs=[
            pl.BlockSpec((br2, 1), lambda i: (i, 0)),
            pl.BlockSpec((br2, 1), lambda i: (i, 0)),
            pl.BlockSpec((nb, 1, ne), lambda i: (0, 0, 0)),
        ],
        out_specs=pl.BlockSpec((br2, ne), lambda i: (i, 0)),
        out_shape=jax.ShapeDtypeStruct((n, ne), jnp.float32),
        compiler_params=pltpu.CompilerParams(
            dimension_semantics=("parallel",)),
    )(pmax, amax, colpart)

    return out


# speedup vs baseline: 1.4036x; 1.0283x over previous
import functools

import jax
import jax.numpy as jnp
from jax.experimental import pallas as pl
from jax.experimental.pallas import tpu as pltpu


def _stage1_body(x_ref, w1t_ref, b1_ref, w2t_ref, b2_ref,
                 pmax_ref, amax_ref, col_ref):
    h = jnp.maximum(
        jnp.dot(x_ref[...], w1t_ref[...], preferred_element_type=jnp.float32)
        + b1_ref[...], 0.0)
    logits = (jnp.dot(h, w2t_ref[...], preferred_element_type=jnp.float32)
              + b2_ref[...])
    br, ne = logits.shape
    m = jnp.max(logits, axis=1, keepdims=True)
    e = jnp.exp(logits - m)
    s = jnp.sum(e, axis=1, keepdims=True)
    pmax = 1.0 / s
    amax = jnp.argmax(logits, axis=1).astype(jnp.int32)[:, None]
    onehot = jax.lax.broadcasted_iota(jnp.int32, (br, ne), 1) == amax
    masked = jnp.where(onehot, pmax, 0.0)
    pmax_ref[...] = pmax
    amax_ref[...] = amax
    col_ref[...] = jnp.sum(masked, axis=0)[None, None, :]


def _stage2_body(pmax_ref, amax_ref, col_ref, out_ref, *, capacity):
    cols = col_ref[...]
    denom = jnp.sum(cols, axis=(0, 1))[None, :] + 0.0001
    t = (pmax_ref[...] / denom) * capacity
    br, ne = t.shape
    onehot = (jax.lax.broadcasted_iota(jnp.int32, (br, ne), 1)
              == amax_ref[...])
    out_ref[...] = jnp.where(onehot, t, 0.0)


def kernel(x, W1, b1, W2, b2):
    n, d = x.shape
    nh = W1.shape[0]
    ne = W2.shape[0]
    br = min(4096, n)
    nb = n // br
    capacity = float(n)

    w1t = W1.T
    w2t = W2.T
    b1r = b1.reshape(1, nh)
    b2r = b2.reshape(1, ne)

    pmax, amax, colpart = pl.pallas_call(
        _stage1_body,
        grid=(nb,),
        in_specs=[
            pl.BlockSpec((br, d), lambda i: (i, 0)),
            pl.BlockSpec((d, nh), lambda i: (0, 0)),
            pl.BlockSpec((1, nh), lambda i: (0, 0)),
            pl.BlockSpec((nh, ne), lambda i: (0, 0)),
            pl.BlockSpec((1, ne), lambda i: (0, 0)),
        ],
        out_specs=[
            pl.BlockSpec((br, 1), lambda i: (i, 0)),
            pl.BlockSpec((br, 1), lambda i: (i, 0)),
            pl.BlockSpec((1, 1, ne), lambda i: (i, 0, 0)),
        ],
        out_shape=[
            jax.ShapeDtypeStruct((n, 1), jnp.float32),
            jax.ShapeDtypeStruct((n, 1), jnp.int32),
            jax.ShapeDtypeStruct((nb, 1, ne), jnp.float32),
        ],
        compiler_params=pltpu.CompilerParams(
            dimension_semantics=("parallel",)),
    )(x, w1t, b1r, w2t, b2r)

    br2 = min(16384, n)
    nb2 = n // br2
    out = pl.pallas_call(
        functools.partial(_stage2_body, capacity=capacity),
        grid=(nb2,),
        in_specs=[
            pl.BlockSpec((br2, 1), lambda i: (i, 0)),
            pl.BlockSpec((br2, 1), lambda i: (i, 0)),
            pl.BlockSpec((nb, 1, ne), lambda i: (0, 0, 0)),
        ],
        out_specs=pl.BlockSpec((br2, ne), lambda i: (i, 0)),
        out_shape=jax.ShapeDtypeStruct((n, ne), jnp.float32),
        compiler_params=pltpu.CompilerParams(
            dimension_semantics=("parallel",)),
    )(pmax, amax, colpart)

    return out


# speedup vs baseline: 1.4039x; 1.0002x over previous
import functools

import jax
import jax.numpy as jnp
from jax.experimental import pallas as pl
from jax.experimental.pallas import tpu as pltpu


def _stage1_body(x_ref, w1t_ref, b1_ref, w2t_ref, b2_ref,
                 pmax_ref, amax_ref, col_ref):
    h = jnp.maximum(
        jnp.dot(x_ref[...], w1t_ref[...], preferred_element_type=jnp.float32)
        + b1_ref[...], 0.0)
    logits = (jnp.dot(h, w2t_ref[...], preferred_element_type=jnp.float32)
              + b2_ref[...])
    br, ne = logits.shape
    m = jnp.max(logits, axis=1, keepdims=True)
    e = jnp.exp(logits - m)
    s = jnp.sum(e, axis=1, keepdims=True)
    pmax = 1.0 / s
    amax = jnp.argmax(logits, axis=1).astype(jnp.int32)[:, None]
    onehot = jax.lax.broadcasted_iota(jnp.int32, (br, ne), 1) == amax
    masked = jnp.where(onehot, pmax, 0.0)
    pmax_ref[...] = pmax
    amax_ref[...] = amax
    col_ref[...] = jnp.sum(masked, axis=0)[None, None, :]


def _stage2_body(pmax_ref, amax_ref, col_ref, out_ref, *, capacity):
    cols = col_ref[...]
    denom = jnp.sum(cols, axis=(0, 1))[None, :] + 0.0001
    t = pmax_ref[...] * (capacity / denom)
    br, ne = t.shape
    onehot = (jax.lax.broadcasted_iota(jnp.int32, (br, ne), 1)
              == amax_ref[...])
    out_ref[...] = jnp.where(onehot, t, 0.0)


def kernel(x, W1, b1, W2, b2):
    n, d = x.shape
    nh = W1.shape[0]
    ne = W2.shape[0]
    br = min(4096, n)
    nb = n // br
    capacity = float(n)

    w1t = W1.T
    w2t = W2.T
    b1r = b1.reshape(1, nh)
    b2r = b2.reshape(1, ne)

    pmax, amax, colpart = pl.pallas_call(
        _stage1_body,
        grid=(nb,),
        in_specs=[
            pl.BlockSpec((br, d), lambda i: (i, 0)),
            pl.BlockSpec((d, nh), lambda i: (0, 0)),
            pl.BlockSpec((1, nh), lambda i: (0, 0)),
            pl.BlockSpec((nh, ne), lambda i: (0, 0)),
            pl.BlockSpec((1, ne), lambda i: (0, 0)),
        ],
        out_specs=[
            pl.BlockSpec((br, 1), lambda i: (i, 0)),
            pl.BlockSpec((br, 1), lambda i: (i, 0)),
            pl.BlockSpec((1, 1, ne), lambda i: (i, 0, 0)),
        ],
        out_shape=[
            jax.ShapeDtypeStruct((n, 1), jnp.float32),
            jax.ShapeDtypeStruct((n, 1), jnp.int32),
            jax.ShapeDtypeStruct((nb, 1, ne), jnp.float32),
        ],
        compiler_params=pltpu.CompilerParams(
            dimension_semantics=("parallel",)),
    )(x, w1t, b1r, w2t, b2r)

    br2 = min(16384, n)
    nb2 = n // br2
    out = pl.pallas_call(
        functools.partial(_stage2_body, capacity=capacity),
        grid=(nb2,),
        in_specs=[
            pl.BlockSpec((br2, 1), lambda i: (i, 0)),
            pl.BlockSpec((br2, 1), lambda i: (i, 0)),
            pl.BlockSpec((nb, 1, ne), lambda i: (0, 0, 0)),
        ],
        out_specs=pl.BlockSpec((br2, ne), lambda i: (i, 0)),
        out_shape=jax.ShapeDtypeStruct((n, ne), jnp.float32),
        compiler_params=pltpu.CompilerParams(
            dimension_semantics=("parallel",)),
    )(pmax, amax, colpart)

    return out
